# Initial kernel scaffold; baseline (speedup 1.0000x reference)
import jax
import jax.numpy as jnp
from jax.experimental import pallas as pl


def kernel(inputs, inputs_mask, another_inputs_full, indexes, back, features, labels, label_weight, label_count):
    raise NotImplementedError("write your pallas kernel here")


# speedup vs baseline: 7.3788x; 7.3788x over previous
import functools

import jax
import jax.numpy as jnp
from jax import lax
from jax.experimental import pallas as pl
from jax.experimental.pallas import tpu as pltpu
from jax.experimental.pallas import tpu_sc as plsc

NUM_SAMPLES = 100000
NUM_FEATURES = 64
NUM_CLASSES = 1000
B = 64
TEMP = 0.05

NC = 2
NS = 16
NW = NC * NS
CPAD = 1024
CHUNK = 80
NCHUNKS = NUM_SAMPLES // CHUNK
SLOTS = NCHUNKS // NW
WCH = 8
FULL_WAVES = SLOTS // WCH
REM = SLOTS - FULL_WAVES * WCH


def _sc_body(feat_hbm, lab_hbm, idx_hbm,
             psum_hbm, cnt_hbm, tgt_hbm,
             featbuf, labbuf, onesv, zbuf, cstage,
             idxv, tgtv,
             acc_sh, cnt_sh,
             load_sem, scat_sem):
    c = lax.axis_index("c")
    s = lax.axis_index("s")
    wid = c * NS + s

    one16 = jnp.ones((16,), jnp.float32)
    zero16 = jnp.zeros((16,), jnp.float32)
    for i in range(CHUNK):
        for j in range(NUM_FEATURES // 16):
            onesv[i, pl.ds(j * 16, 16)] = one16
    for i in range(64):
        for j in range(NUM_FEATURES // 16):
            zbuf[i, pl.ds(j * 16, 16)] = zero16

    pltpu.sync_copy(zbuf, acc_sh.at[pl.ds(s * 64, 64), :])
    pltpu.sync_copy(zbuf, cnt_sh.at[pl.ds(s * 64, 64), :])
    plsc.subcore_barrier()

    def run_wave(w, nch):
        loads = []
        for k in range(nch):
            base = (wid + (w * WCH + k) * NW) * CHUNK
            loads.append(pltpu.async_copy(
                feat_hbm.at[pl.ds(base, CHUNK), :],
                featbuf.at[pl.ds(k * CHUNK, CHUNK), :], load_sem))
            loads.append(pltpu.async_copy(
                lab_hbm.at[pl.ds(base, CHUNK)], labbuf.at[k], load_sem))
        for d in loads:
            d.wait()
        scats = []
        for k in range(nch):
            scats.append(pltpu.async_copy(
                featbuf.at[pl.ds(k * CHUNK, CHUNK), :],
                acc_sh.at[labbuf.at[k]], scat_sem, add=True))
            scats.append(pltpu.async_copy(
                onesv, cnt_sh.at[labbuf.at[k]], scat_sem, add=True))
        for d in scats:
            d.wait()

    def wave(w, carry):
        run_wave(w, WCH)
        return carry

    lax.fori_loop(0, FULL_WAVES, wave, 0)
    run_wave(FULL_WAVES, REM)

    @pl.when(wid < NCHUNKS - SLOTS * NW)
    def _tail():
        base = (SLOTS * NW + wid) * CHUNK
        pltpu.sync_copy(feat_hbm.at[pl.ds(base, CHUNK), :],
                        featbuf.at[pl.ds(0, CHUNK), :])
        pltpu.sync_copy(lab_hbm.at[pl.ds(base, CHUNK)], labbuf.at[0])
        pltpu.sync_copy(featbuf.at[pl.ds(0, CHUNK), :],
                        acc_sh.at[labbuf.at[0]], add=True)
        pltpu.sync_copy(onesv, cnt_sh.at[labbuf.at[0]], add=True)

    plsc.subcore_barrier()

    pltpu.sync_copy(acc_sh.at[pl.ds(s * 64, 64), :], zbuf)
    pltpu.sync_copy(zbuf, psum_hbm.at[c, pl.ds(s * 64, 64), :])
    pltpu.sync_copy(cnt_sh.at[pl.ds(s * 64, 64), :], cstage)
    pltpu.sync_copy(cstage, cnt_hbm.at[c, pl.ds(s * 64, 64), :])

    @pl.when(jnp.logical_and(c == 0, s == 0))
    def _tgt():
        pltpu.sync_copy(idx_hbm, idxv)
        pltpu.async_copy(lab_hbm.at[idxv], tgtv, load_sem).wait()
        pltpu.sync_copy(tgtv, tgt_hbm)


@functools.cache
def _make_sc_segment_sums():
  return pl.kernel(
    _sc_body,
    out_type=(
        jax.ShapeDtypeStruct((NC, CPAD, NUM_FEATURES), jnp.float32),
        jax.ShapeDtypeStruct((NC, CPAD, NUM_FEATURES), jnp.float32),
        jax.ShapeDtypeStruct((B,), jnp.int32),
    ),
    mesh=plsc.VectorSubcoreMesh(core_axis_name="c", subcore_axis_name="s"),
    scratch_types=[
        pltpu.VMEM((WCH * CHUNK, NUM_FEATURES), jnp.float32),
        pltpu.VMEM((WCH, CHUNK), jnp.int32),
        pltpu.VMEM((CHUNK, NUM_FEATURES), jnp.float32),
        pltpu.VMEM((64, NUM_FEATURES), jnp.float32),
        pltpu.VMEM((64, NUM_FEATURES), jnp.float32),
        pltpu.VMEM((B,), jnp.int32),
        pltpu.VMEM((B,), jnp.int32),
        pltpu.VMEM_SHARED((CPAD, NUM_FEATURES), jnp.float32),
        pltpu.VMEM_SHARED((CPAD, NUM_FEATURES), jnp.float32),
        pltpu.SemaphoreType.DMA,
        pltpu.SemaphoreType.DMA,
    ],
  )


def _tc_body(psum_ref, cnt_ref, inputs_ref, another_ref, tgt_ref, out_ref):
    S = psum_ref[0] + psum_ref[1]
    n = (cnt_ref[0] + cnt_ref[1])[:, 0:1]
    mask = (n > 0.0).astype(jnp.float32)
    md = S / (mask * n + (1.0 - mask))

    sim = lax.dot_general(
        md, inputs_ref[...], (((1,), (1,)), ((), ())),
        preferred_element_type=jnp.float32,
        precision=lax.Precision.HIGHEST) * (1.0 / TEMP)

    e = jnp.exp(sim) * mask
    denom = jnp.sum(e, axis=0, keepdims=True) + 1e-6
    cls = lax.broadcasted_iota(jnp.int32, (CPAD, B), 0)
    onehot = (cls == tgt_ref[...]).astype(jnp.float32)
    p = jnp.sum(e * onehot, axis=0, keepdims=True) / denom
    floss = jnp.sum(-((1.0 - p) ** 4) * jnp.log(p + 1e-6)) / B

    inp = lax.dot_general(
        onehot, md, (((0,), (0,)), ((), ())),
        preferred_element_type=jnp.float32,
        precision=lax.Precision.HIGHEST)
    inp = inp / jnp.sqrt(jnp.sum(inp * inp, axis=1, keepdims=True))
    another = another_ref[...]
    another = another / jnp.sqrt(jnp.sum(another * another, axis=1, keepdims=True))
    cml = -jnp.sum(inp * another) / B

    ninp = inputs_ref[...]
    ninp = ninp / jnp.sqrt(jnp.sum(ninp * ninp, axis=1, keepdims=True))
    cl = -jnp.sum(ninp * another) / B

    full = floss + cml + cl
    lane = lax.broadcasted_iota(jnp.int32, (1, 128), 1)
    out_ref[...] = (jnp.where(lane == 0, floss, 0.0)
                    + jnp.where(lane == 1, full, 0.0))


_tc_epilogue = pl.pallas_call(
    _tc_body,
    out_shape=jax.ShapeDtypeStruct((1, 128), jnp.float32),
)


def kernel(inputs, inputs_mask, another_inputs_full, indexes, back,
           features, labels, label_weight, label_count):
    del inputs_mask, label_weight, label_count
    psum, cnt, tgt = _make_sc_segment_sums()(
        features, labels.astype(jnp.int32), indexes.astype(jnp.int32))
    out = _tc_epilogue(psum, cnt, inputs, another_inputs_full,
                       tgt.reshape(1, B))
    floss = out[0, 0]
    full_loss = out[0, 1]
    return jnp.where(back == 0, floss, full_loss)
